# baseline (speedup 1.0000x reference)
import functools
import numpy as np
import jax
import jax.numpy as jnp
from jax import lax
from jax.experimental import pallas as pl
from jax.experimental.pallas import tpu as pltpu
from jax.experimental.pallas import tpu_sc as plsc

_B, _S, _D, _V = 4096, 50, 128, 1000
_NW = 32
_CB = _B // _NW
_HB = _CB // 2
_NCH = _S * 2
_NBUF = 10
_AHEAD = 4


def _pos_enc(seq_len, d_model):
    pos = np.arange(seq_len)[:, None].astype(np.float32)
    i = np.arange(d_model)[None, :].astype(np.float32)
    angle = pos / np.power(10000.0, (2.0 * np.floor(i / 2.0)) / d_model)
    pe = np.zeros((seq_len, d_model), dtype=np.float32)
    pe[:, 0::2] = np.sin(angle[:, 0::2])
    pe[:, 1::2] = np.cos(angle[:, 1::2])
    return pe


_mesh = plsc.VectorSubcoreMesh(core_axis_name="c", subcore_axis_name="s")


@functools.partial(
    pl.kernel,
    out_type=jax.ShapeDtypeStruct((_S, _B, _D), jnp.float32),
    mesh=_mesh,
    compiler_params=pltpu.CompilerParams(use_tc_tiling_on_sc=True),
    scratch_types=[
        pltpu.VMEM((_S, _CB), jnp.int32),
        pltpu.VMEM((_S * 4, _CB), jnp.float32),
        pltpu.VMEM((_NBUF, _HB, _D), jnp.float32),
        pltpu.VMEM((_S * _D,), jnp.float32),
        pltpu.VMEM((4 * _D,), jnp.float32),
        pltpu.SemaphoreType.DMA((_NBUF,)),
        pltpu.SemaphoreType.DMA((_NBUF,)),
    ],
)
def _sc_kernel(label_h, bbox_h, table_h, wt_h, peb_h, out_h,
               idx_v, bb_v, rows_v, pe_v, w_v, sem_g, sem_o):
    cid = lax.axis_index("c")
    sid = lax.axis_index("s")
    wid = sid * 2 + cid
    b0w = wid * _CB
    pltpu.sync_copy(wt_h, w_v)
    pltpu.sync_copy(peb_h, pe_v)
    pltpu.sync_copy(label_h.at[:, pl.ds(b0w, _CB)], idx_v)
    pltpu.sync_copy(bbox_h.at[:, pl.ds(b0w, _CB)], bb_v)

    def load_wv():
        return [[w_v[pl.ds(f * _D + dc * 16, 16)] for f in range(4)]
                for dc in range(8)]

    def start_gather(c, half, slot):
        pltpu.async_copy(
            table_h.at[idx_v.at[c // 2, pl.ds(half * _HB, _HB)]],
            rows_v.at[slot], sem_g.at[slot])

    def wait_gather(slot):
        pltpu.make_async_copy(table_h.at[idx_v.at[0, pl.ds(0, _HB)]],
                              rows_v.at[slot], sem_g.at[slot]).wait()

    def start_writeback(c, half, slot):
        pltpu.async_copy(
            rows_v.at[slot],
            out_h.at[c // 2].at[pl.ds(b0w + half * _HB, _HB)],
            sem_o.at[slot])

    def drain_writeback(slot):
        pltpu.make_async_copy(rows_v.at[slot],
                              out_h.at[0].at[pl.ds(b0w, _HB)],
                              sem_o.at[slot]).wait()

    def compute(c, half, slot):
        si = c // 2
        pes = [pe_v[pl.ds(si * _D + dc * 16, 16)] for dc in range(8)]

        def tok16(tg, c2):
            Wv = load_wv()
            t0 = tg * 16
            bbf = [bb_v[si * 4 + f, pl.ds(half * _HB + t0, 16)]
                   for f in range(4)]
            for ti in range(16):
                b0f = bbf[0][ti]
                b1f = bbf[1][ti]
                b2f = bbf[2][ti]
                b3f = bbf[3][ti]
                t = t0 + ti
                for dc in range(8):
                    d0 = dc * 16
                    acc = rows_v[slot, t, pl.ds(d0, 16)] + pes[dc]
                    acc = acc + b0f * Wv[dc][0] + b1f * Wv[dc][1]
                    acc = acc + b2f * Wv[dc][2] + b3f * Wv[dc][3]
                    rows_v[slot, t, pl.ds(d0, 16)] = acc
            return c2

        lax.fori_loop(0, _HB // 16, tok16, 0)

    for c in range(_AHEAD):
        start_gather(c, c % 2, c)

    def body(g, carry):
        s = lax.rem(g, _NBUF)
        half = lax.rem(g, 2)

        @pl.when(g <= _NCH - 1 - _AHEAD)
        def _():
            h = lax.rem(g + _AHEAD, _NBUF)

            @pl.when(g >= _NBUF - _AHEAD)
            def _():
                drain_writeback(h)
            start_gather(g + _AHEAD, lax.rem(g + _AHEAD, 2), h)

        wait_gather(s)
        compute(g, half, s)
        start_writeback(g, half, s)
        return carry

    lax.fori_loop(0, _NCH, body, 0)

    for s in range(_NBUF):
        drain_writeback(s)


def kernel(label, bbox, label_table, W_bbox, b_bbox):
    label_t = jnp.transpose(label).astype(jnp.int32)
    bb_t = jnp.transpose(bbox, (1, 2, 0)).reshape(_S * 4, _B)
    wt = jnp.transpose(W_bbox).reshape(4 * _D)
    peb = (jnp.asarray(_pos_enc(_S, _D)) + b_bbox[None, :]).reshape(_S * _D)
    out = _sc_kernel(label_t, bb_t, label_table, wt, peb)
    return jnp.transpose(out, (1, 0, 2))


# speedup vs baseline: 1.4105x; 1.4105x over previous
import functools
import numpy as np
import jax
import jax.numpy as jnp
from jax import lax
from jax.experimental import pallas as pl
from jax.experimental.pallas import tpu as pltpu
from jax.experimental.pallas import tpu_sc as plsc

_B, _S, _D, _V = 4096, 50, 128, 1000
_NW = 32
_CB = _B // _NW
_HB = _CB // 2
_NCH = _S * 2
_NBUF = 2


def _pos_enc(seq_len, d_model):
    pos = np.arange(seq_len)[:, None].astype(np.float32)
    i = np.arange(d_model)[None, :].astype(np.float32)
    angle = pos / np.power(10000.0, (2.0 * np.floor(i / 2.0)) / d_model)
    pe = np.zeros((seq_len, d_model), dtype=np.float32)
    pe[:, 0::2] = np.sin(angle[:, 0::2])
    pe[:, 1::2] = np.cos(angle[:, 1::2])
    return pe


_mesh = plsc.VectorSubcoreMesh(core_axis_name="c", subcore_axis_name="s")


@functools.partial(
    pl.kernel,
    out_type=jax.ShapeDtypeStruct((_S, _B, _D), jnp.float32),
    mesh=_mesh,
    compiler_params=pltpu.CompilerParams(use_tc_tiling_on_sc=True,
                                         needs_layout_passes=False),
    scratch_types=[
        pltpu.VMEM((_S, _CB), jnp.int32),
        pltpu.VMEM((_S * 4, _CB), jnp.float32),
        pltpu.VMEM((_V * _D // 2,), jnp.int32),
        pltpu.VMEM((_NBUF, _HB, _D), jnp.float32),
        pltpu.VMEM((_S * _D,), jnp.float32),
        pltpu.VMEM((4 * _D,), jnp.float32),
        pltpu.SemaphoreType.DMA((_NBUF,)),
    ],
)
def _sc_kernel(label_h, bbox_h, table_h, wt_h, peb_h, out_h,
               idx_v, bb_v, tab_v, obuf, pe_v, w_v, sem_o):
    cid = lax.axis_index("c")
    sid = lax.axis_index("s")
    wid = sid * 2 + cid
    b0w = wid * _CB
    pltpu.sync_copy(wt_h, w_v)
    pltpu.sync_copy(peb_h, pe_v)
    pltpu.sync_copy(table_h, tab_v)
    pltpu.sync_copy(label_h.at[:, pl.ds(b0w, _CB)], idx_v)
    pltpu.sync_copy(bbox_h.at[:, pl.ds(b0w, _CB)], bb_v)

    def load_wv():
        return [[w_v[pl.ds(f * _D + dc * 16, 16)] for f in range(4)]
                for dc in range(8)]

    def start_writeback(c, half, slot):
        pltpu.async_copy(
            obuf.at[slot],
            out_h.at[c // 2].at[pl.ds(b0w + half * _HB, _HB)],
            sem_o.at[slot])

    def drain_writeback(slot):
        pltpu.make_async_copy(obuf.at[slot],
                              out_h.at[0].at[pl.ds(b0w, _HB)],
                              sem_o.at[slot]).wait()

    def compute(c, half, slot):
        si = c // 2
        pes = [pe_v[pl.ds(si * _D + dc * 16, 16)] for dc in range(8)]

        def tok16(tg, c2):
            Wv = load_wv()
            t0 = tg * 16
            idx16 = idx_v[si, pl.ds(half * _HB + t0, 16)]
            bbf = [bb_v[si * 4 + f, pl.ds(half * _HB + t0, 16)]
                   for f in range(4)]
            for ti in range(16):
                lab = idx16[ti]
                base = pl.multiple_of(lab * (_D // 2), _D // 2)
                b0f = bbf[0][ti]
                b1f = bbf[1][ti]
                b2f = bbf[2][ti]
                b3f = bbf[3][ti]
                t = t0 + ti
                for gk in range(4):
                    pk32 = tab_v[pl.ds(base + gk * 16, 16)]
                    pk = plsc.bitcast(pk32, jnp.bfloat16)
                    ev, od = plsc.unpack(
                        pk, format=plsc.PackFormat.INTERLEAVED)
                    for dc, vec in ((2 * gk, ev), (2 * gk + 1, od)):
                        acc = vec + pes[dc]
                        acc = acc + b0f * Wv[dc][0] + b1f * Wv[dc][1]
                        acc = acc + b2f * Wv[dc][2] + b3f * Wv[dc][3]
                        obuf[slot, t, pl.ds(dc * 16, 16)] = acc
            return c2

        lax.fori_loop(0, _HB // 16, tok16, 0)

    def super_body(go, carry):
        for kslot in range(_NBUF):
            g = go * _NBUF + kslot
            half = kslot

            @pl.when(g >= _NBUF)
            def _():
                drain_writeback(kslot)
            compute(g, half, kslot)
            start_writeback(g, half, kslot)
        return carry

    lax.fori_loop(0, _NCH // _NBUF, super_body, 0)

    for s in range(_NBUF):
        drain_writeback(s)


def kernel(label, bbox, label_table, W_bbox, b_bbox):
    label_t = jnp.transpose(label).astype(jnp.int32)
    bb_t = jnp.transpose(bbox, (1, 2, 0)).reshape(_S * 4, _B)
    t4 = label_table.reshape(_V, 4, 2, 16)
    inter = jnp.stack([t4[:, :, 0, :], t4[:, :, 1, :]], axis=-1)
    table_bf = (inter.astype(jnp.bfloat16)
                .reshape(_V * _D // 2, 2))
    table_i32 = lax.bitcast_convert_type(table_bf, jnp.int32)
    wt = jnp.transpose(W_bbox).reshape(4 * _D)
    peb = (jnp.asarray(_pos_enc(_S, _D)) + b_bbox[None, :]).reshape(_S * _D)
    out = _sc_kernel(label_t, bb_t, table_i32, wt, peb)
    return jnp.transpose(out, (1, 0, 2))


# speedup vs baseline: 2.9731x; 2.1077x over previous
import functools
import numpy as np
import jax
import jax.numpy as jnp
from jax import lax
from jax.experimental import pallas as pl
from jax.experimental.pallas import tpu as pltpu
from jax.experimental.pallas import tpu_sc as plsc

_B, _S, _D, _V = 4096, 50, 128, 1000
_NW = 32
_CB = _B // _NW
_NBUF = 5


def _pos_enc(seq_len, d_model):
    pos = np.arange(seq_len)[:, None].astype(np.float32)
    i = np.arange(d_model)[None, :].astype(np.float32)
    angle = pos / np.power(10000.0, (2.0 * np.floor(i / 2.0)) / d_model)
    pe = np.zeros((seq_len, d_model), dtype=np.float32)
    pe[:, 0::2] = np.sin(angle[:, 0::2])
    pe[:, 1::2] = np.cos(angle[:, 1::2])
    return pe


_mesh = plsc.VectorSubcoreMesh(core_axis_name="c", subcore_axis_name="s")


@functools.partial(
    pl.kernel,
    out_type=jax.ShapeDtypeStruct((_S, _B, _D), jnp.float32),
    mesh=_mesh,
    compiler_params=pltpu.CompilerParams(use_tc_tiling_on_sc=True),
    scratch_types=[
        pltpu.VMEM((_S, _CB), jnp.int32),
        pltpu.VMEM((_S * 4, _CB), jnp.float32),
        pltpu.VMEM((_NBUF, _CB, _D), jnp.float32),
        pltpu.VMEM((_S * _D,), jnp.float32),
        pltpu.VMEM((4 * _D,), jnp.float32),
        pltpu.SemaphoreType.DMA((_NBUF,)),
        pltpu.SemaphoreType.DMA((_NBUF,)),
    ],
)
def _sc_kernel(label_h, bbox_h, table_h, wt_h, peb_h, out_h,
               idx_v, bb_v, rows_v, pe_v, w_v, sem_g, sem_o):
    cid = lax.axis_index("c")
    sid = lax.axis_index("s")
    wid = sid * 2 + cid
    b0w = wid * _CB
    pltpu.sync_copy(wt_h, w_v)
    pltpu.sync_copy(peb_h, pe_v)
    pltpu.sync_copy(label_h.at[:, pl.ds(b0w, _CB)], idx_v)
    pltpu.sync_copy(bbox_h.at[:, pl.ds(b0w, _CB)], bb_v)

    Wv = [[w_v[pl.ds(f * _D + dc * 16, 16)] for f in range(4)]
          for dc in range(8)]

    def start_gather(c, slot):
        pltpu.async_copy(table_h.at[idx_v.at[c]], rows_v.at[slot],
                         sem_g.at[slot])

    def wait_gather(slot):
        pltpu.make_async_copy(table_h.at[idx_v.at[0]], rows_v.at[slot],
                              sem_g.at[slot]).wait()

    def start_writeback(c, slot):
        pltpu.async_copy(rows_v.at[slot], out_h.at[c].at[pl.ds(b0w, _CB)],
                         sem_o.at[slot])

    def drain_writeback(slot):
        pltpu.make_async_copy(rows_v.at[slot],
                              out_h.at[0].at[pl.ds(b0w, _CB)],
                              sem_o.at[slot]).wait()

    def compute(c, slot):
        pes = [pe_v[pl.ds(c * _D + dc * 16, 16)] for dc in range(8)]

        def tok16(tg, c2):
            t0 = tg * 16
            bbf = [bb_v[c * 4 + f, pl.ds(t0, 16)] for f in range(4)]
            for ti in range(16):
                b0f = bbf[0][ti]
                b1f = bbf[1][ti]
                b2f = bbf[2][ti]
                b3f = bbf[3][ti]
                t = t0 + ti
                for dc in range(8):
                    d0 = dc * 16
                    acc = rows_v[slot, t, pl.ds(d0, 16)] + pes[dc]
                    acc = acc + b0f * Wv[dc][0] + b1f * Wv[dc][1]
                    acc = acc + b2f * Wv[dc][2] + b3f * Wv[dc][3]
                    rows_v[slot, t, pl.ds(d0, 16)] = acc
            return c2

        lax.fori_loop(0, _CB // 16, tok16, 0)

    start_gather(0, 0)
    start_gather(1, 1)

    def super_body(go, carry):
        for kslot in range(_NBUF):
            g = go * _NBUF + kslot
            s = kslot

            @pl.when(g <= _S - 3)
            def _():
                h = (s + 2) % _NBUF

                @pl.when(g >= _NBUF - 2)
                def _():
                    drain_writeback(h)
                start_gather(g + 2, h)

            wait_gather(s)
            compute(g, s)
            start_writeback(g, s)
        return carry

    lax.fori_loop(0, _S // _NBUF, super_body, 0)

    for s in range(_NBUF):
        drain_writeback(s)


def kernel(label, bbox, label_table, W_bbox, b_bbox):
    label_t = jnp.transpose(label).astype(jnp.int32)
    bb_t = jnp.transpose(bbox, (1, 2, 0)).reshape(_S * 4, _B)
    wt = jnp.transpose(W_bbox).reshape(4 * _D)
    peb = (jnp.asarray(_pos_enc(_S, _D)) + b_bbox[None, :]).reshape(_S * _D)
    out = _sc_kernel(label_t, bb_t, label_table, wt, peb)
    return jnp.transpose(out, (1, 0, 2))
